# baseline (speedup 1.0000x reference)
import functools

import jax
import jax.numpy as jnp
from jax import lax
from jax.experimental import pallas as pl
from jax.experimental.pallas import tpu as pltpu
from jax.experimental.pallas import tpu_sc as plsc

N = 10000
E = 320000
F_IN = 128
HEADS = 8
HID = 16
D = HEADS * HID

NC = 2
NS = 16
NW = NC * NS
B = 64
CH = 32
NBLK = 160
B2 = 128
NBLK2 = 80
CH2 = 16
EPW = NBLK * B
EPAD = EPW * NW
NPAD = 10240
ROWS_PT = NPAD // NS

_BM = 1024


def _pre_body(x_ref, w_ref, ms_ref, md_ref, h_ref, as_ref, ad_ref):
    y = jnp.dot(x_ref[...], w_ref[...], preferred_element_type=jnp.float32)
    h_ref[...] = y
    as_ref[...] = jnp.dot(y, ms_ref[...], preferred_element_type=jnp.float32)
    ad_ref[...] = jnp.dot(y, md_ref[...], preferred_element_type=jnp.float32)


def _dense_pre(x, W, Ms, Md):
    grid = (NPAD // _BM,)
    return pl.pallas_call(
        _pre_body,
        grid=grid,
        in_specs=[
            pl.BlockSpec((_BM, F_IN), lambda i: (i, 0)),
            pl.BlockSpec((F_IN, D), lambda i: (0, 0)),
            pl.BlockSpec((D, D), lambda i: (0, 0)),
            pl.BlockSpec((D, D), lambda i: (0, 0)),
        ],
        out_specs=[
            pl.BlockSpec((_BM, D), lambda i: (i, 0)),
            pl.BlockSpec((_BM, D), lambda i: (i, 0)),
            pl.BlockSpec((_BM, D), lambda i: (i, 0)),
        ],
        out_shape=[
            jax.ShapeDtypeStruct((NPAD, D), jnp.float32),
            jax.ShapeDtypeStruct((NPAD, D), jnp.float32),
            jax.ShapeDtypeStruct((NPAD, D), jnp.float32),
        ],
    )(x, W, Ms, Md)


def _combine(h_ref, as_ref, ad_ref, msg_ref, den_ref, b_ref, kill_ref):
    t = as_ref[...] + ad_ref[...]
    wself = jnp.exp(jnp.maximum(t, 0.2 * t))
    wrep = jnp.dot(wself, kill_ref[...], preferred_element_type=jnp.float32)
    denrep = den_ref[0] + den_ref[1] + wrep
    msg = msg_ref[0] + msg_ref[1] + h_ref[...] * wrep
    return msg / (denrep + 1e-16) + b_ref[...]


def _mid_body(h_ref, as_ref, ad_ref, msg_ref, den_ref, b_ref,
              kill_ref, w2_ref, ms_ref, md_ref, h2_ref, as2_ref, ad2_ref):
    z = jnp.maximum(_combine(h_ref, as_ref, ad_ref, msg_ref, den_ref,
                             b_ref, kill_ref), 0.0)
    y = jnp.dot(z, w2_ref[...], preferred_element_type=jnp.float32)
    h2_ref[...] = y
    as2_ref[...] = jnp.dot(y, ms_ref[...], preferred_element_type=jnp.float32)
    ad2_ref[...] = jnp.dot(y, md_ref[...], preferred_element_type=jnp.float32)


def _dense_mid(h, AS, AD, msg, den, b, Kill, W2, Ms2, Md2):
    grid = (NPAD // _BM,)
    return pl.pallas_call(
        _mid_body,
        grid=grid,
        in_specs=[
            pl.BlockSpec((_BM, D), lambda i: (i, 0)),
            pl.BlockSpec((_BM, D), lambda i: (i, 0)),
            pl.BlockSpec((_BM, D), lambda i: (i, 0)),
            pl.BlockSpec((2, _BM, D), lambda i: (0, i, 0)),
            pl.BlockSpec((2, _BM, D), lambda i: (0, i, 0)),
            pl.BlockSpec((1, D), lambda i: (0, 0)),
            pl.BlockSpec((D, D), lambda i: (0, 0)),
            pl.BlockSpec((D, D), lambda i: (0, 0)),
            pl.BlockSpec((D, D), lambda i: (0, 0)),
            pl.BlockSpec((D, D), lambda i: (0, 0)),
        ],
        out_specs=[
            pl.BlockSpec((_BM, D), lambda i: (i, 0)),
            pl.BlockSpec((_BM, D), lambda i: (i, 0)),
            pl.BlockSpec((_BM, D), lambda i: (i, 0)),
        ],
        out_shape=[
            jax.ShapeDtypeStruct((NPAD, D), jnp.float32),
            jax.ShapeDtypeStruct((NPAD, D), jnp.float32),
            jax.ShapeDtypeStruct((NPAD, D), jnp.float32),
        ],
    )(h, AS, AD, msg, den, b, Kill, W2, Ms2, Md2)


def _post_body(h_ref, as_ref, ad_ref, msg_ref, den_ref, b_ref, kill_ref,
               out_ref):
    out_ref[...] = _combine(h_ref, as_ref, ad_ref, msg_ref, den_ref,
                            b_ref, kill_ref)


def _dense_post(h, AS, AD, msg, den, b, Kill):
    grid = (NPAD // _BM,)
    return pl.pallas_call(
        _post_body,
        grid=grid,
        in_specs=[
            pl.BlockSpec((_BM, D), lambda i: (i, 0)),
            pl.BlockSpec((_BM, D), lambda i: (i, 0)),
            pl.BlockSpec((_BM, D), lambda i: (i, 0)),
            pl.BlockSpec((2, _BM, D), lambda i: (0, i, 0)),
            pl.BlockSpec((2, _BM, D), lambda i: (0, i, 0)),
            pl.BlockSpec((1, D), lambda i: (0, 0)),
            pl.BlockSpec((D, D), lambda i: (0, 0)),
        ],
        out_specs=pl.BlockSpec((_BM, D), lambda i: (i, 0)),
        out_shape=jax.ShapeDtypeStruct((NPAD, D), jnp.float32),
    )(h, AS, AD, msg, den, b, Kill)


def _sc1_body(h_hbm, as_hbm, ad_hbm, src_hbm, dst_hbm,
              msg_out, w_out,
              scb, dcb, g1a, g2a, gha, wva, macc,
              semA1, semA2, semA3):
    c = lax.axis_index("c")
    s = lax.axis_index("s")
    wid = s * NC + c
    row0 = s * ROWS_PT

    def _zb(e, carry):
        for k in range(HEADS):
            gha[e, pl.ds(16 * k, 16)] = jnp.zeros((16,), jnp.float32)
        return carry
    lax.fori_loop(0, B, _zb, None)

    def _zcp(k, carry):
        pltpu.sync_copy(gha, macc.at[pl.ds(row0 + k * B, B)])
        return carry
    lax.fori_loop(0, ROWS_PT // B, _zcp, None)
    plsc.subcore_barrier()

    def _do_block(g1, g2, gh, wvm, didx, blk):
        def _edge(e, _c):
            t = g1[e, pl.ds(0, 16)] + g2[e, pl.ds(0, 16)]
            w = jnp.exp(jnp.maximum(t, 0.2 * t))
            wvm[e] = w
            for k in range(HEADS):
                wk = jnp.full((16,), w[k], jnp.float32)
                gh[e, pl.ds(16 * k, 16)] = gh[e, pl.ds(16 * k, 16)] * wk
            return _c
        lax.fori_loop(0, B, _edge, None, unroll=2)
        pltpu.sync_copy(gh, macc.at[didx], add=True)
        pltpu.sync_copy(wvm, w_out.at[pl.ds(blk * B, B)])

    def _chunk(cc, carry):
        cbase = wid * NBLK + cc * CH
        pltpu.sync_copy(src_hbm.at[pl.ds(cbase, CH)], scb)
        pltpu.sync_copy(dst_hbm.at[pl.ds(cbase, CH)], dcb)

        def _block(jj, carry2):
            sidx, didx = scb.at[jj], dcb.at[jj]
            cp1 = pltpu.async_copy(as_hbm.at[sidx], g1a, semA1)
            cp2 = pltpu.async_copy(ad_hbm.at[didx], g2a, semA2)
            cp3 = pltpu.async_copy(h_hbm.at[sidx], gha, semA3)
            cp1.wait()
            cp2.wait()
            cp3.wait()
            _do_block(g1a, g2a, gha, wva, didx, cbase + jj)
            return carry2

        lax.fori_loop(0, CH, _block, None)
        return carry

    lax.fori_loop(0, NBLK // CH, _chunk, None)
    plsc.subcore_barrier()

    def _drain(k, carry):
        r = row0 + k * B
        pltpu.sync_copy(macc.at[pl.ds(r, B)], gha)
        pltpu.sync_copy(gha, msg_out.at[pl.ds(c * NPAD + r, B)])
        return carry
    lax.fori_loop(0, ROWS_PT // B, _drain, None)


_sc_msg = functools.partial(
    pl.kernel,
    mesh=plsc.VectorSubcoreMesh(core_axis_name="c", subcore_axis_name="s"),
    out_type=[
        jax.ShapeDtypeStruct((NC * NPAD, D), jnp.float32),
        jax.ShapeDtypeStruct((EPAD, 16), jnp.float32),
    ],
    scratch_types=[
        pltpu.VMEM((CH, B), jnp.int32),
        pltpu.VMEM((CH, B), jnp.int32),
        pltpu.VMEM((B, D), jnp.float32),
        pltpu.VMEM((B, D), jnp.float32),
        pltpu.VMEM((B, D), jnp.float32),
        pltpu.VMEM((B, 16), jnp.float32),
        pltpu.VMEM_SHARED((NPAD, D), jnp.float32),
        pltpu.SemaphoreType.DMA,
        pltpu.SemaphoreType.DMA,
        pltpu.SemaphoreType.DMA,
    ],
)(_sc1_body)


def _sc2_body(w_hbm, dst_hbm, den_out, dcb, wvm, wrow, dacc):
    c = lax.axis_index("c")
    s = lax.axis_index("s")
    wid = s * NC + c
    row0 = s * ROWS_PT

    def _zb(e, carry):
        for k in range(HEADS):
            wrow[e, pl.ds(16 * k, 16)] = jnp.zeros((16,), jnp.float32)
        return carry
    lax.fori_loop(0, B2, _zb, None)

    def _zcp(k, carry):
        pltpu.sync_copy(wrow, dacc.at[pl.ds(row0 + k * B2, B2)])
        return carry
    lax.fori_loop(0, ROWS_PT // B2, _zcp, None)
    plsc.subcore_barrier()

    def _chunk(cc, carry):
        cbase = wid * NBLK2 + cc * CH2
        pltpu.sync_copy(dst_hbm.at[pl.ds(cbase, CH2)], dcb)

        def _block(jj, carry2):
            didx = dcb.at[jj]
            pltpu.sync_copy(w_hbm.at[pl.ds((cbase + jj) * B2, B2)], wvm)

            def _edge(e, _c):
                w = wvm[e]
                for k in range(HEADS):
                    wk = jnp.full((16,), w[k], jnp.float32)
                    wrow[e, pl.ds(16 * k, 16)] = wk
                return _c
            lax.fori_loop(0, B2, _edge, None)

            pltpu.sync_copy(wrow, dacc.at[didx], add=True)
            return carry2

        lax.fori_loop(0, CH2, _block, None)
        return carry

    lax.fori_loop(0, NBLK2 // CH2, _chunk, None)
    plsc.subcore_barrier()

    def _drain(k, carry):
        r = row0 + k * B2
        pltpu.sync_copy(dacc.at[pl.ds(r, B2)], wrow)
        pltpu.sync_copy(wrow, den_out.at[pl.ds(c * NPAD + r, B2)])
        return carry
    lax.fori_loop(0, ROWS_PT // B2, _drain, None)


_sc_den = functools.partial(
    pl.kernel,
    mesh=plsc.VectorSubcoreMesh(core_axis_name="c", subcore_axis_name="s"),
    out_type=jax.ShapeDtypeStruct((NC * NPAD, D), jnp.float32),
    scratch_types=[
        pltpu.VMEM((CH2, B2), jnp.int32),
        pltpu.VMEM((B2, 16), jnp.float32),
        pltpu.VMEM((B2, D), jnp.float32),
        pltpu.VMEM_SHARED((NPAD, D), jnp.float32),
    ],
)(_sc2_body)


def _head_mat(att):
    m = jnp.zeros((D, D), jnp.float32)
    rows = jnp.arange(D)
    cols = jnp.repeat(jnp.arange(HEADS), HID)
    return m.at[rows, cols].set(att.reshape(-1))


def kernel(x, edge_index, W1, att_src1, att_dst1, b1,
           W2, att_src2, att_dst2, b2):
    x_pad = jnp.pad(x, ((0, NPAD - N), (0, 0)))
    pad = jnp.full((EPAD - E,), N, jnp.int32)
    src_r = jnp.concatenate([edge_index[0], pad]).reshape(NW * NBLK, B)
    dst_r = jnp.concatenate([edge_index[1], pad]).reshape(NW * NBLK, B)
    dst_r2 = dst_r.reshape(NW * NBLK2, B2)

    Ms1, Md1 = _head_mat(att_src1), _head_mat(att_dst1)
    Ms2, Md2 = _head_mat(att_src2), _head_mat(att_dst2)
    hrep = jnp.kron(jnp.eye(8, dtype=jnp.float32),
                    jnp.ones((1, HID), jnp.float32))
    kill = jnp.zeros((D, D), jnp.float32).at[:HEADS].set(hrep)
    b1r = b1.reshape(1, D)
    b2r = b2.reshape(1, D)

    h1, AS1, AD1 = _dense_pre(x_pad, W1, Ms1, Md1)
    msg1, w1e = _sc_msg(h1, AS1, AD1, src_r, dst_r)
    den1 = _sc_den(w1e, dst_r2)
    h2, AS2, AD2 = _dense_mid(h1, AS1, AD1, msg1.reshape(NC, NPAD, D),
                              den1.reshape(NC, NPAD, D), b1r, kill,
                              W2, Ms2, Md2)
    msg2, w2e = _sc_msg(h2, AS2, AD2, src_r, dst_r)
    den2 = _sc_den(w2e, dst_r2)
    z = _dense_post(h2, AS2, AD2, msg2.reshape(NC, NPAD, D),
                    den2.reshape(NC, NPAD, D), b2r, kill)
    return z[:N]


# speedup vs baseline: 1.1155x; 1.1155x over previous
import functools

import jax
import jax.numpy as jnp
from jax import lax
from jax.experimental import pallas as pl
from jax.experimental.pallas import tpu as pltpu
from jax.experimental.pallas import tpu_sc as plsc

N = 10000
E = 320000
F_IN = 128
HEADS = 8
HID = 16
D = HEADS * HID

NC = 2
NS = 16
NW = NC * NS
B = 64
CH = 16
NBLK = 160
B2 = 128
NBLK2 = 80
CH2 = 16
EPW = NBLK * B
EPAD = EPW * NW
NPAD = 10240
ROWS_PT = NPAD // NS

_BM = 1024


def _pre_body(x_ref, w_ref, ms_ref, md_ref, h_ref, as_ref, ad_ref):
    y = jnp.dot(x_ref[...], w_ref[...], preferred_element_type=jnp.float32)
    h_ref[...] = y
    as_ref[...] = jnp.dot(y, ms_ref[...], preferred_element_type=jnp.float32)
    ad_ref[...] = jnp.dot(y, md_ref[...], preferred_element_type=jnp.float32)


def _dense_pre(x, W, Ms, Md):
    grid = (NPAD // _BM,)
    return pl.pallas_call(
        _pre_body,
        grid=grid,
        in_specs=[
            pl.BlockSpec((_BM, F_IN), lambda i: (i, 0)),
            pl.BlockSpec((F_IN, D), lambda i: (0, 0)),
            pl.BlockSpec((D, D), lambda i: (0, 0)),
            pl.BlockSpec((D, D), lambda i: (0, 0)),
        ],
        out_specs=[
            pl.BlockSpec((_BM, D), lambda i: (i, 0)),
            pl.BlockSpec((_BM, D), lambda i: (i, 0)),
            pl.BlockSpec((_BM, D), lambda i: (i, 0)),
        ],
        out_shape=[
            jax.ShapeDtypeStruct((NPAD, D), jnp.float32),
            jax.ShapeDtypeStruct((NPAD, D), jnp.float32),
            jax.ShapeDtypeStruct((NPAD, D), jnp.float32),
        ],
    )(x, W, Ms, Md)


def _combine(h_ref, as_ref, ad_ref, msg_ref, den_ref, b_ref, kill_ref):
    t = as_ref[...] + ad_ref[...]
    wself = jnp.exp(jnp.maximum(t, 0.2 * t))
    wrep = jnp.dot(wself, kill_ref[...], preferred_element_type=jnp.float32)
    denrep = den_ref[0] + den_ref[1] + wrep
    msg = msg_ref[0] + msg_ref[1] + h_ref[...] * wrep
    return msg / (denrep + 1e-16) + b_ref[...]


def _mid_body(h_ref, as_ref, ad_ref, msg_ref, den_ref, b_ref,
              kill_ref, w2_ref, ms_ref, md_ref, h2_ref, as2_ref, ad2_ref):
    z = jnp.maximum(_combine(h_ref, as_ref, ad_ref, msg_ref, den_ref,
                             b_ref, kill_ref), 0.0)
    y = jnp.dot(z, w2_ref[...], preferred_element_type=jnp.float32)
    h2_ref[...] = y
    as2_ref[...] = jnp.dot(y, ms_ref[...], preferred_element_type=jnp.float32)
    ad2_ref[...] = jnp.dot(y, md_ref[...], preferred_element_type=jnp.float32)


def _dense_mid(h, AS, AD, msg, den, b, Kill, W2, Ms2, Md2):
    grid = (NPAD // _BM,)
    return pl.pallas_call(
        _mid_body,
        grid=grid,
        in_specs=[
            pl.BlockSpec((_BM, D), lambda i: (i, 0)),
            pl.BlockSpec((_BM, D), lambda i: (i, 0)),
            pl.BlockSpec((_BM, D), lambda i: (i, 0)),
            pl.BlockSpec((2, _BM, D), lambda i: (0, i, 0)),
            pl.BlockSpec((2, _BM, D), lambda i: (0, i, 0)),
            pl.BlockSpec((1, D), lambda i: (0, 0)),
            pl.BlockSpec((D, D), lambda i: (0, 0)),
            pl.BlockSpec((D, D), lambda i: (0, 0)),
            pl.BlockSpec((D, D), lambda i: (0, 0)),
            pl.BlockSpec((D, D), lambda i: (0, 0)),
        ],
        out_specs=[
            pl.BlockSpec((_BM, D), lambda i: (i, 0)),
            pl.BlockSpec((_BM, D), lambda i: (i, 0)),
            pl.BlockSpec((_BM, D), lambda i: (i, 0)),
        ],
        out_shape=[
            jax.ShapeDtypeStruct((NPAD, D), jnp.float32),
            jax.ShapeDtypeStruct((NPAD, D), jnp.float32),
            jax.ShapeDtypeStruct((NPAD, D), jnp.float32),
        ],
    )(h, AS, AD, msg, den, b, Kill, W2, Ms2, Md2)


def _post_body(h_ref, as_ref, ad_ref, msg_ref, den_ref, b_ref, kill_ref,
               out_ref):
    out_ref[...] = _combine(h_ref, as_ref, ad_ref, msg_ref, den_ref,
                            b_ref, kill_ref)


def _dense_post(h, AS, AD, msg, den, b, Kill):
    grid = (NPAD // _BM,)
    return pl.pallas_call(
        _post_body,
        grid=grid,
        in_specs=[
            pl.BlockSpec((_BM, D), lambda i: (i, 0)),
            pl.BlockSpec((_BM, D), lambda i: (i, 0)),
            pl.BlockSpec((_BM, D), lambda i: (i, 0)),
            pl.BlockSpec((2, _BM, D), lambda i: (0, i, 0)),
            pl.BlockSpec((2, _BM, D), lambda i: (0, i, 0)),
            pl.BlockSpec((1, D), lambda i: (0, 0)),
            pl.BlockSpec((D, D), lambda i: (0, 0)),
        ],
        out_specs=pl.BlockSpec((_BM, D), lambda i: (i, 0)),
        out_shape=jax.ShapeDtypeStruct((NPAD, D), jnp.float32),
    )(h, AS, AD, msg, den, b, Kill)


def _sc1_body(h_hbm, as_hbm, ad_hbm, src_hbm, dst_hbm,
              msg_out, w_out,
              scb, dcb, g1a, g2a, gha, wva, macc,
              semA1, semA2, semA3):
    c = lax.axis_index("c")
    s = lax.axis_index("s")
    wid = s * NC + c
    row0 = s * ROWS_PT

    def _zb(e, carry):
        for k in range(HEADS):
            gha[e, pl.ds(16 * k, 16)] = jnp.zeros((16,), jnp.float32)
        return carry
    lax.fori_loop(0, B, _zb, None)

    def _zcp(k, carry):
        pltpu.sync_copy(gha, macc.at[pl.ds(row0 + k * B, B)])
        return carry
    lax.fori_loop(0, ROWS_PT // B, _zcp, None)
    plsc.subcore_barrier()

    def _do_block(g1, g2, gh, wvm, didx, blk):
        def _edge(e, _c):
            t = g1[e, pl.ds(0, 16)] + g2[e, pl.ds(0, 16)]
            w = jnp.exp(jnp.maximum(t, 0.2 * t))
            wvm[e] = w
            for k in range(HEADS):
                wk = jnp.full((16,), w[k], jnp.float32)
                gh[e, pl.ds(16 * k, 16)] = gh[e, pl.ds(16 * k, 16)] * wk
            return _c
        lax.fori_loop(0, B, _edge, None)
        pltpu.sync_copy(gh, macc.at[didx], add=True)
        pltpu.sync_copy(wvm, w_out.at[pl.ds(blk * B, B)])

    def _chunk(cc, carry):
        cbase = wid * NBLK + cc * CH
        pltpu.sync_copy(src_hbm.at[pl.ds(cbase, CH)], scb)
        pltpu.sync_copy(dst_hbm.at[pl.ds(cbase, CH)], dcb)

        def _block(jj, carry2):
            sidx, didx = scb.at[jj], dcb.at[jj]
            cp1 = pltpu.async_copy(as_hbm.at[sidx], g1a, semA1)
            cp2 = pltpu.async_copy(ad_hbm.at[didx], g2a, semA2)
            cp3 = pltpu.async_copy(h_hbm.at[sidx], gha, semA3)
            cp1.wait()
            cp2.wait()
            cp3.wait()
            _do_block(g1a, g2a, gha, wva, didx, cbase + jj)
            return carry2

        lax.fori_loop(0, CH, _block, None)
        return carry

    lax.fori_loop(0, NBLK // CH, _chunk, None)
    plsc.subcore_barrier()

    def _drain(k, carry):
        r = row0 + k * B
        pltpu.sync_copy(macc.at[pl.ds(r, B)], gha)
        pltpu.sync_copy(gha, msg_out.at[pl.ds(c * NPAD + r, B)])
        return carry
    lax.fori_loop(0, ROWS_PT // B, _drain, None)


_sc_msg = functools.partial(
    pl.kernel,
    mesh=plsc.VectorSubcoreMesh(core_axis_name="c", subcore_axis_name="s"),
    out_type=[
        jax.ShapeDtypeStruct((NC * NPAD, D), jnp.float32),
        jax.ShapeDtypeStruct((EPAD, 16), jnp.float32),
    ],
    scratch_types=[
        pltpu.VMEM((CH, B), jnp.int32),
        pltpu.VMEM((CH, B), jnp.int32),
        pltpu.VMEM((B, D), jnp.float32),
        pltpu.VMEM((B, D), jnp.float32),
        pltpu.VMEM((B, D), jnp.float32),
        pltpu.VMEM((B, 16), jnp.float32),
        pltpu.VMEM_SHARED((NPAD, D), jnp.float32),
        pltpu.SemaphoreType.DMA,
        pltpu.SemaphoreType.DMA,
        pltpu.SemaphoreType.DMA,
    ],
)(_sc1_body)


def _sc2_body(w_hbm, dst_hbm, den_out, dcb, wvm, wrow, dacc):
    c = lax.axis_index("c")
    s = lax.axis_index("s")
    wid = s * NC + c
    row0 = s * ROWS_PT

    def _zb(e, carry):
        for k in range(HEADS):
            wrow[e, pl.ds(16 * k, 16)] = jnp.zeros((16,), jnp.float32)
        return carry
    lax.fori_loop(0, B2, _zb, None)

    def _zcp(k, carry):
        pltpu.sync_copy(wrow, dacc.at[pl.ds(row0 + k * B2, B2)])
        return carry
    lax.fori_loop(0, ROWS_PT // B2, _zcp, None)
    plsc.subcore_barrier()

    def _chunk(cc, carry):
        cbase = wid * NBLK2 + cc * CH2
        pltpu.sync_copy(dst_hbm.at[pl.ds(cbase, CH2)], dcb)

        def _block(jj, carry2):
            didx = dcb.at[jj]
            pltpu.sync_copy(w_hbm.at[pl.ds((cbase + jj) * B2, B2)], wvm)

            def _edge(e, _c):
                w = wvm[e]
                for k in range(HEADS):
                    wk = jnp.full((16,), w[k], jnp.float32)
                    wrow[e, pl.ds(16 * k, 16)] = wk
                return _c
            lax.fori_loop(0, B2, _edge, None)

            pltpu.sync_copy(wrow, dacc.at[didx], add=True)
            return carry2

        lax.fori_loop(0, CH2, _block, None)
        return carry

    lax.fori_loop(0, NBLK2 // CH2, _chunk, None)
    plsc.subcore_barrier()

    def _drain(k, carry):
        r = row0 + k * B2
        pltpu.sync_copy(dacc.at[pl.ds(r, B2)], wrow)
        pltpu.sync_copy(wrow, den_out.at[pl.ds(c * NPAD + r, B2)])
        return carry
    lax.fori_loop(0, ROWS_PT // B2, _drain, None)


_sc_den = functools.partial(
    pl.kernel,
    mesh=plsc.VectorSubcoreMesh(core_axis_name="c", subcore_axis_name="s"),
    out_type=jax.ShapeDtypeStruct((NC * NPAD, D), jnp.float32),
    scratch_types=[
        pltpu.VMEM((CH2, B2), jnp.int32),
        pltpu.VMEM((B2, 16), jnp.float32),
        pltpu.VMEM((B2, D), jnp.float32),
        pltpu.VMEM_SHARED((NPAD, D), jnp.float32),
    ],
)(_sc2_body)


def _head_mat(att):
    m = jnp.zeros((D, D), jnp.float32)
    rows = jnp.arange(D)
    cols = jnp.repeat(jnp.arange(HEADS), HID)
    return m.at[rows, cols].set(att.reshape(-1))


def kernel(x, edge_index, W1, att_src1, att_dst1, b1,
           W2, att_src2, att_dst2, b2):
    x_pad = jnp.pad(x, ((0, NPAD - N), (0, 0)))
    pad = jnp.full((EPAD - E,), N, jnp.int32)
    src_r = jnp.concatenate([edge_index[0], pad]).reshape(NW * NBLK, B)
    dst_r = jnp.concatenate([edge_index[1], pad]).reshape(NW * NBLK, B)
    dst_r2 = dst_r.reshape(NW * NBLK2, B2)

    Ms1, Md1 = _head_mat(att_src1), _head_mat(att_dst1)
    Ms2, Md2 = _head_mat(att_src2), _head_mat(att_dst2)
    hrep = jnp.kron(jnp.eye(8, dtype=jnp.float32),
                    jnp.ones((1, HID), jnp.float32))
    kill = jnp.zeros((D, D), jnp.float32).at[:HEADS].set(hrep)
    b1r = b1.reshape(1, D)
    b2r = b2.reshape(1, D)

    h1, AS1, AD1 = _dense_pre(x_pad, W1, Ms1, Md1)
    msg1, w1e = _sc_msg(h1, AS1, AD1, src_r, dst_r)
    den1 = _sc_den(w1e, dst_r2)
    h2, AS2, AD2 = _dense_mid(h1, AS1, AD1, msg1.reshape(NC, NPAD, D),
                              den1.reshape(NC, NPAD, D), b1r, kill,
                              W2, Ms2, Md2)
    msg2, w2e = _sc_msg(h2, AS2, AD2, src_r, dst_r)
    den2 = _sc_den(w2e, dst_r2)
    z = _dense_post(h2, AS2, AD2, msg2.reshape(NC, NPAD, D),
                    den2.reshape(NC, NPAD, D), b2r, kill)
    return z[:N]
